# baseline (speedup 1.0000x reference)
import functools

import jax
import jax.numpy as jnp
from jax import lax
from jax.experimental import pallas as pl
from jax.experimental.pallas import tpu as pltpu
from jax.experimental.pallas import tpu_sc as plsc

_N = 10000
_E = 320000
_D = 128

_NC = 2
_NS = 16
_NW = _NC * _NS
_C = 64
_NCHUNK = 160
_EPW = _C * _NCHUNK
_EP = _EPW * _NW
_NP = 10240
_DUMMY = 10000
_RPT = _NP // _NS
_ZROWS = 128
_DEPTH = 3


def _sc_scatter(src3, dst3, x, zeros):
    mesh = plsc.VectorSubcoreMesh(core_axis_name="c", subcore_axis_name="s")

    @functools.partial(
        pl.kernel,
        mesh=mesh,
        out_type=jax.ShapeDtypeStruct((_NC, _NP, _D), jnp.float32),
        scratch_types=(
            [pltpu.VMEM((_NCHUNK, _C), jnp.int32)] +
            [pltpu.VMEM((_C,), jnp.int32)] * _DEPTH +
            [pltpu.VMEM((_C, _D), jnp.float32)] * _DEPTH +
            [pltpu.VMEM_SHARED((_NP, _D), jnp.float32)] +
            [pltpu.SemaphoreType.DMA] * (3 * _DEPTH + 1)
        ),
    )
    def k(src_hbm, dst_hbm, x_hbm, z_hbm, out_hbm, sidx,
          d0, d1, d2, r0, r1, r2, acc,
          g0, g1, g2, s0, s1, s2, f0, f1, f2, zs):
        didx = [d0, d1, d2]
        rows = [r0, r1, r2]
        gs = [g0, g1, g2]
        ss = [s0, s1, s2]
        fs = [f0, f1, f2]
        cid = lax.axis_index("c")
        sid = lax.axis_index("s")
        wid = sid * _NC + cid
        base_row = sid * _RPT

        def dfetch(j, k_):
            return pltpu.async_copy(dst_hbm.at[wid, j], didx[k_], fs[k_])

        def gather(j, k_):
            return pltpu.async_copy(x_hbm.at[sidx.at[j]], rows[k_], gs[k_])

        def scat(k_):
            return pltpu.async_copy(rows[k_], acc.at[didx[k_]], ss[k_],
                                    add=True)

        gw = [pltpu.make_async_copy(x_hbm.at[didx[k_]], rows[k_], gs[k_])
              for k_ in range(_DEPTH)]
        sw = [pltpu.make_async_copy(rows[k_], acc.at[didx[k_]], ss[k_])
              for k_ in range(_DEPTH)]
        dw = [pltpu.make_async_copy(dst_hbm.at[wid, 0], didx[k_], fs[k_])
              for k_ in range(_DEPTH)]

        zcps = [pltpu.async_copy(
            z_hbm, acc.at[pl.ds(base_row + j * _ZROWS, _ZROWS)], zs)
            for j in range(_RPT // _ZROWS)]
        pltpu.sync_copy(src_hbm.at[wid], sidx)
        for j in range(_DEPTH):
            dfetch(j, j)
            gather(j, j)
        for z in zcps:
            z.wait()
        plsc.subcore_barrier()
        gw[0].wait()
        dw[0].wait()
        scat(0)
        gw[1].wait()
        dw[1].wait()
        scat(1)

        def trip(i, carry):
            for u in range(_DEPTH):
                j = _DEPTH * i + _DEPTH + u
                k_ = u
                p = (u + _DEPTH - 1) % _DEPTH
                sw[k_].wait()
                dfetch(j, k_)
                gather(j, k_)
                gw[p].wait()
                dw[p].wait()
                scat(p)
            return carry
        lax.fori_loop(0, (_NCHUNK - _DEPTH - 1) // _DEPTH, trip, 0)

        sw[0].wait()
        dfetch(_NCHUNK - 1, 0)
        gather(_NCHUNK - 1, 0)
        gw[2].wait()
        dw[2].wait()
        scat(2)
        gw[0].wait()
        dw[0].wait()
        scat(0)
        sw[1].wait()
        sw[2].wait()
        sw[0].wait()
        plsc.subcore_barrier()

        pltpu.sync_copy(acc.at[pl.ds(base_row, _RPT)],
                        out_hbm.at[cid, pl.ds(base_row, _RPT)])

    return k(src3, dst3, x, zeros)


def _tc_add(p0, p1):
    blk = 1000

    def body(a_ref, b_ref, o_ref):
        o_ref[...] = a_ref[...] + b_ref[...]

    return pl.pallas_call(
        body,
        grid=(_N // blk,),
        in_specs=[pl.BlockSpec((blk, _D), lambda i: (i, 0)),
                  pl.BlockSpec((blk, _D), lambda i: (i, 0))],
        out_specs=pl.BlockSpec((blk, _D), lambda i: (i, 0)),
        out_shape=jax.ShapeDtypeStruct((_N, _D), jnp.float32),
    )(p0, p1)


def kernel(edge_index, x):
    ei = edge_index.astype(jnp.int32)
    npad = _EP - _E
    src = jnp.concatenate([ei[0], jnp.zeros((npad,), jnp.int32)])
    dst = jnp.concatenate([ei[1], jnp.full((npad,), _DUMMY, jnp.int32)])
    src3 = src.reshape(_NW, _NCHUNK, _C)
    dst3 = dst.reshape(_NW, _NCHUNK, _C)
    zeros = jnp.zeros((_ZROWS, _D), jnp.float32)
    partials = _sc_scatter(src3, dst3, x, zeros)
    return _tc_add(partials[0], partials[1])


# speedup vs baseline: 3.3456x; 3.3456x over previous
import functools

import jax
import jax.numpy as jnp
from jax import lax
from jax.experimental import pallas as pl
from jax.experimental.pallas import tpu as pltpu
from jax.experimental.pallas import tpu_sc as plsc

_N = 10000
_E = 320000
_D = 128

_NC = 2
_NS = 16
_NW = _NC * _NS
_C = 64
_NCHUNK = 160
_EPW = _C * _NCHUNK
_EP = _EPW * _NW
_NP = 10240
_DUMMY = 10000
_RPT = _NP // _NS
_ZROWS = 128
_DEPTH = 3


def _sc_scatter(src3, dst3, x, zeros):
    mesh = plsc.VectorSubcoreMesh(core_axis_name="c", subcore_axis_name="s")

    @functools.partial(
        pl.kernel,
        mesh=mesh,
        out_type=jax.ShapeDtypeStruct((_NC, _NP, _D), jnp.float32),
        scratch_types=(
            [pltpu.VMEM((_NCHUNK, _C), jnp.int32)] +
            [pltpu.VMEM((_C,), jnp.int32)] * _DEPTH +
            [pltpu.VMEM((_C, _D), jnp.float32)] * _DEPTH +
            [pltpu.VMEM_SHARED((_NP, _D), jnp.float32)] +
            [pltpu.SemaphoreType.DMA] * (3 * _DEPTH + 1)
        ),
    )
    def k(src_hbm, dst_hbm, x_hbm, z_hbm, out_hbm, sidx,
          d0, d1, d2, r0, r1, r2, acc,
          g0, g1, g2, s0, s1, s2, f0, f1, f2, zs):
        didx = [d0, d1, d2]
        rows = [r0, r1, r2]
        gs = [g0, g1, g2]
        ss = [s0, s1, s2]
        fs = [f0, f1, f2]
        cid = lax.axis_index("c")
        sid = lax.axis_index("s")
        wid = sid * _NC + cid
        base_row = sid * _RPT

        def dfetch(j, k_):
            return pltpu.async_copy(dst_hbm.at[wid, j], didx[k_], fs[k_])

        def gather(j, k_):
            return pltpu.async_copy(x_hbm.at[sidx.at[j]], rows[k_], gs[k_])

        def scat(k_):
            return pltpu.async_copy(rows[k_], acc.at[didx[k_]], ss[k_],
                                    add=True)

        gw = [pltpu.make_async_copy(x_hbm.at[didx[k_]], rows[k_], gs[k_])
              for k_ in range(_DEPTH)]
        sw = [pltpu.make_async_copy(rows[k_], acc.at[didx[k_]], ss[k_])
              for k_ in range(_DEPTH)]
        dw = [pltpu.make_async_copy(dst_hbm.at[wid, 0], didx[k_], fs[k_])
              for k_ in range(_DEPTH)]

        zcps = [pltpu.async_copy(
            z_hbm, acc.at[pl.ds(base_row + j * _ZROWS, _ZROWS)], zs)
            for j in range(_RPT // _ZROWS)]
        pltpu.sync_copy(src_hbm.at[wid], sidx)
        for j in range(_DEPTH):
            dfetch(j, j)
            gather(j, j)
        for z in zcps:
            z.wait()
        plsc.subcore_barrier()
        gw[0].wait()
        dw[0].wait()
        scat(0)
        gw[1].wait()
        dw[1].wait()
        scat(1)

        def trip(i, carry):
            for u in range(_DEPTH):
                j = _DEPTH * i + _DEPTH + u
                k_ = u
                p = (u + _DEPTH - 1) % _DEPTH
                sw[k_].wait()
                dfetch(j, k_)
                gather(j, k_)
                gw[p].wait()
                dw[p].wait()
                scat(p)
            return carry
        lax.fori_loop(0, (_NCHUNK - _DEPTH - 1) // _DEPTH, trip, 0)

        sw[0].wait()
        dfetch(_NCHUNK - 1, 0)
        gather(_NCHUNK - 1, 0)
        gw[2].wait()
        dw[2].wait()
        scat(2)
        gw[0].wait()
        dw[0].wait()
        scat(0)
        sw[1].wait()
        sw[2].wait()
        sw[0].wait()
        plsc.subcore_barrier()

        pltpu.sync_copy(acc.at[pl.ds(base_row, _RPT)],
                        out_hbm.at[cid, pl.ds(base_row, _RPT)])

    return k(src3, dst3, x, zeros)


def _tc_add(p0, p1):
    blk = 1000

    def body(a_ref, b_ref, o_ref):
        o_ref[...] = a_ref[...] + b_ref[...]

    return pl.pallas_call(
        body,
        grid=(_N // blk,),
        in_specs=[pl.BlockSpec((blk, _D), lambda i: (i, 0)),
                  pl.BlockSpec((blk, _D), lambda i: (i, 0))],
        out_specs=pl.BlockSpec((blk, _D), lambda i: (i, 0)),
        out_shape=jax.ShapeDtypeStruct((_N, _D), jnp.float32),
    )(p0, p1)


def kernel(edge_index, x):
    ei = edge_index.astype(jnp.int32)
    npad = _EP - _E
    pad_iota = jnp.arange(npad, dtype=jnp.int32)
    src = jnp.concatenate([ei[0], pad_iota % _N])
    dst = jnp.concatenate([ei[1], _DUMMY + pad_iota % (_NP - _DUMMY)])
    src3 = src.reshape(_NW, _NCHUNK, _C)
    dst3 = dst.reshape(_NW, _NCHUNK, _C)
    zeros = jnp.zeros((_ZROWS, _D), jnp.float32)
    partials = _sc_scatter(src3, dst3, x, zeros)
    return _tc_add(partials[0], partials[1])
